# Initial kernel scaffold; baseline (speedup 1.0000x reference)
import jax
import jax.numpy as jnp
from jax.experimental import pallas as pl


def kernel(x, edge_index, W1, b1, Wmu, bmu, Wlv, blv, eps):
    raise NotImplementedError("write your pallas kernel here")


# speedup vs baseline: 20.8215x; 20.8215x over previous
import functools

import jax
import jax.numpy as jnp
from jax import lax
from jax.experimental import pallas as pl
from jax.experimental.pallas import tpu as pltpu
from jax.experimental.pallas import tpu_sc as plsc

N = 10000
E = 320000
NF = 128
NH = 64
NL = 32
ADJ = 2000

C = 128
NCHUNK = E // C
NWORK = 32
CPW = -(-NCHUNK // NWORK)
NSUB = 16
NP = 10240
RPS = NP // NSUB
DEGW = 8

_MESH = plsc.VectorSubcoreMesh(core_axis_name="c", subcore_axis_name="s")
_SC_PARAMS = pltpu.CompilerParams(use_tc_tiling_on_sc=False)


@functools.partial(
    pl.kernel,
    mesh=_MESH,
    out_type=jax.ShapeDtypeStruct((2, NP, DEGW), jnp.float32),
    scratch_types=[
        pltpu.VMEM((C,), jnp.int32),
        pltpu.VMEM((C, DEGW), jnp.float32),
        pltpu.VMEM_SHARED((NP, DEGW), jnp.float32),
    ],
    compiler_params=_SC_PARAMS,
)
def _sc_degree(dst_hbm, ones_hbm, zeros_hbm, out_hbm, didx, ones_v, acc):
    c = lax.axis_index("c")
    s = lax.axis_index("s")
    wid = s * 2 + c
    row = s * RPS
    pltpu.sync_copy(zeros_hbm.at[pl.ds(row, RPS)], acc.at[pl.ds(row, RPS)])
    pltpu.sync_copy(ones_hbm, ones_v)
    plsc.subcore_barrier()

    def step(k, carry):
        chunk = wid + NWORK * k

        @pl.when(chunk < NCHUNK)
        def _():
            pltpu.sync_copy(dst_hbm.at[pl.ds(chunk * C, C)], didx)
            pltpu.sync_copy(ones_v, acc.at[didx], add=True)

        return carry

    lax.fori_loop(0, CPW, step, 0)
    plsc.subcore_barrier()
    pltpu.sync_copy(acc.at[pl.ds(row, RPS)], out_hbm.at[c, pl.ds(row, RPS)])


@functools.partial(
    pl.kernel,
    mesh=_MESH,
    out_type=jax.ShapeDtypeStruct((2, NP, NH), jnp.float32),
    scratch_types=[
        pltpu.VMEM((C,), jnp.int32),
        pltpu.VMEM((C,), jnp.int32),
        pltpu.VMEM((C, NH), jnp.float32),
        pltpu.VMEM_SHARED((NP, NH), jnp.float32),
        pltpu.SemaphoreType.DMA,
    ],
    compiler_params=_SC_PARAMS,
)
def _sc_pass(table_hbm, src_hbm, dst_hbm, zeros_hbm, out_hbm,
             sidx, didx, rows, acc, sem):
    c = lax.axis_index("c")
    s = lax.axis_index("s")
    wid = s * 2 + c
    row = s * RPS
    pltpu.sync_copy(zeros_hbm.at[pl.ds(row, RPS)], acc.at[pl.ds(row, RPS)])
    plsc.subcore_barrier()

    def step(k, carry):
        chunk = wid + NWORK * k

        @pl.when(chunk < NCHUNK)
        def _():
            base = chunk * C
            pltpu.sync_copy(src_hbm.at[pl.ds(base, C)], sidx)
            pltpu.sync_copy(dst_hbm.at[pl.ds(base, C)], didx)
            pltpu.async_copy(table_hbm.at[sidx], rows, sem).wait()
            pltpu.sync_copy(rows, acc.at[didx], add=True)

        return carry

    lax.fori_loop(0, CPW, step, 0)
    plsc.subcore_barrier()
    pltpu.sync_copy(acc.at[pl.ds(row, RPS)], out_hbm.at[c, pl.ds(row, RPS)])


_B = 1000


def _dinv_from(dcnt):
    deg = dcnt[0, :, 0] + dcnt[1, :, 0] + 1.0
    return lax.rsqrt(deg)[:, None]


def _xs_body(x_ref, w1_ref, dcnt_ref, xs_ref):
    dinv = _dinv_from(dcnt_ref[...])
    xs_ref[...] = dinv * jnp.dot(x_ref[...], w1_ref[...],
                                 preferred_element_type=jnp.float32)


def _tc_xs(x, w1, dcnt):
    return pl.pallas_call(
        _xs_body,
        grid=(N // _B,),
        in_specs=[
            pl.BlockSpec((_B, NF), lambda i: (i, 0)),
            pl.BlockSpec((NF, NH), lambda i: (0, 0)),
            pl.BlockSpec((2, _B, DEGW), lambda i: (0, i, 0)),
        ],
        out_specs=pl.BlockSpec((_B, NH), lambda i: (i, 0)),
        out_shape=jax.ShapeDtypeStruct((N, NH), jnp.float32),
    )(x, w1, dcnt)


def _hs_body(p_ref, xs_ref, dcnt_ref, b1_ref, wc_ref, hs_ref):
    dinv = _dinv_from(dcnt_ref[...])
    h = dinv * (p_ref[0] + p_ref[1] + xs_ref[...]) + b1_ref[...][None, :]
    hs_ref[...] = dinv * jnp.dot(h, wc_ref[...],
                                 preferred_element_type=jnp.float32)


def _tc_hs(p1, xs, dcnt, b1, wc):
    return pl.pallas_call(
        _hs_body,
        grid=(N // _B,),
        in_specs=[
            pl.BlockSpec((2, _B, NH), lambda i: (0, i, 0)),
            pl.BlockSpec((_B, NH), lambda i: (i, 0)),
            pl.BlockSpec((2, _B, DEGW), lambda i: (0, i, 0)),
            pl.BlockSpec((NH,), lambda i: (0,)),
            pl.BlockSpec((NH, NH), lambda i: (0, 0)),
        ],
        out_specs=pl.BlockSpec((_B, NH), lambda i: (i, 0)),
        out_shape=jax.ShapeDtypeStruct((N, NH), jnp.float32),
    )(p1, xs, dcnt, b1, wc)


def _z_body(p_ref, hs_ref, dcnt_ref, bmu_ref, blv_ref, eps_ref,
            z_ref, mu_ref, lv_ref):
    dinv = _dinv_from(dcnt_ref[...])
    g2 = dinv * (p_ref[0] + p_ref[1] + hs_ref[...])
    mu = g2[:, :NL] + bmu_ref[...][None, :]
    lv = g2[:, NL:] + blv_ref[...][None, :]
    mu_ref[...] = mu
    lv_ref[...] = lv
    z_ref[...] = mu + eps_ref[...] * jnp.exp(0.5 * lv)


def _tc_z(p2, hs, dcnt, bmu, blv, eps):
    return pl.pallas_call(
        _z_body,
        grid=(N // _B,),
        in_specs=[
            pl.BlockSpec((2, _B, NH), lambda i: (0, i, 0)),
            pl.BlockSpec((_B, NH), lambda i: (i, 0)),
            pl.BlockSpec((2, _B, DEGW), lambda i: (0, i, 0)),
            pl.BlockSpec((NL,), lambda i: (0,)),
            pl.BlockSpec((NL,), lambda i: (0,)),
            pl.BlockSpec((_B, NL), lambda i: (i, 0)),
        ],
        out_specs=[
            pl.BlockSpec((_B, NL), lambda i: (i, 0)),
            pl.BlockSpec((_B, NL), lambda i: (i, 0)),
            pl.BlockSpec((_B, NL), lambda i: (i, 0)),
        ],
        out_shape=[
            jax.ShapeDtypeStruct((N, NL), jnp.float32),
            jax.ShapeDtypeStruct((N, NL), jnp.float32),
            jax.ShapeDtypeStruct((N, NL), jnp.float32),
        ],
    )(p2, hs, dcnt, bmu, blv, eps)


def _zm_body(z_ref, zm_ref):
    z = z_ref[...]
    acc = z[0 * ADJ:1 * ADJ] + z[1 * ADJ:2 * ADJ] + z[2 * ADJ:3 * ADJ]
    acc = acc + z[3 * ADJ:4 * ADJ] + z[4 * ADJ:5 * ADJ]
    zm_ref[...] = acc * (1.0 / (N // ADJ))


def _tc_zm(z):
    return pl.pallas_call(
        _zm_body,
        out_shape=jax.ShapeDtypeStruct((ADJ, NL), jnp.float32),
    )(z)


_BA = 1000


def _adj_body(zi_ref, zj_ref, adj_ref):
    prod = lax.dot_general(zi_ref[...], zj_ref[...],
                           (((1,), (1,)), ((), ())),
                           preferred_element_type=jnp.float32)
    adj_ref[...] = jax.nn.sigmoid(prod)


def _tc_adj(zm):
    return pl.pallas_call(
        _adj_body,
        grid=(ADJ // _BA,),
        in_specs=[
            pl.BlockSpec((_BA, NL), lambda i: (i, 0)),
            pl.BlockSpec((ADJ, NL), lambda i: (0, 0)),
        ],
        out_specs=pl.BlockSpec((_BA, ADJ), lambda i: (i, 0)),
        out_shape=jax.ShapeDtypeStruct((ADJ, ADJ), jnp.float32),
    )(zm, zm)


def kernel(x, edge_index, W1, b1, Wmu, bmu, Wlv, blv, eps):
    src = edge_index[0]
    dst = edge_index[1]
    zeros_deg = jnp.zeros((NP, DEGW), jnp.float32)
    ones_c = jnp.ones((C, DEGW), jnp.float32)
    zeros_h = jnp.zeros((NP, NH), jnp.float32)
    wc = jnp.concatenate([Wmu, Wlv], axis=1)

    dcnt = _sc_degree(dst, ones_c, zeros_deg)
    xs = _tc_xs(x, W1, dcnt)
    p1 = _sc_pass(xs, src, dst, zeros_h)
    hs = _tc_hs(p1, xs, dcnt, b1, wc)
    p2 = _sc_pass(hs, src, dst, zeros_h)
    z, mu, lv = _tc_z(p2, hs, dcnt, bmu, blv, eps)
    zm = _tc_zm(z)
    adj = _tc_adj(zm)
    return (z, mu, lv, adj)


# speedup vs baseline: 37.5652x; 1.8042x over previous
import functools

import jax
import jax.numpy as jnp
from jax import lax
from jax.experimental import pallas as pl
from jax.experimental.pallas import tpu as pltpu
from jax.experimental.pallas import tpu_sc as plsc

N = 10000
E = 320000
NF = 128
NH = 64
NL = 32
ADJ = 2000

C = 128
NCHUNK = E // C
NWORK = 32
NRING = 6
CPW = NCHUNK // NWORK
OUTER = CPW // NRING
TAIL = NCHUNK - NWORK * CPW
NSUB = 16
NP = 10240
RPS = NP // NSUB
DEGW = 8

_MESH = plsc.VectorSubcoreMesh(core_axis_name="c", subcore_axis_name="s")
_SC_PARAMS = pltpu.CompilerParams(use_tc_tiling_on_sc=False)


@functools.partial(
    pl.kernel,
    mesh=_MESH,
    out_type=jax.ShapeDtypeStruct((2, NP, DEGW), jnp.float32),
    scratch_types=[
        pltpu.VMEM((NRING, C), jnp.int32),
        pltpu.VMEM((C, DEGW), jnp.float32),
        pltpu.VMEM_SHARED((NP, DEGW), jnp.float32),
        pltpu.SemaphoreType.DMA((NRING,)),
        pltpu.SemaphoreType.DMA((NRING,)),
    ],
    compiler_params=_SC_PARAMS,
)
def _sc_degree(dst_hbm, ones_hbm, zeros_hbm, out_hbm, didx, ones_v, acc,
               sem_i, sem_s):
    c = lax.axis_index("c")
    s = lax.axis_index("s")
    wid = s * 2 + c
    row = s * RPS
    pltpu.sync_copy(zeros_hbm.at[pl.ds(row, RPS)], acc.at[pl.ds(row, RPS)])
    pltpu.sync_copy(ones_hbm, ones_v)
    plsc.subcore_barrier()

    def round_(j, carry):
        di = []
        for b in range(NRING):
            base = (wid + NWORK * (j * NRING + b)) * C
            di.append(pltpu.async_copy(
                dst_hbm.at[pl.ds(base, C)], didx.at[b], sem_i.at[b]))
        sc = []
        for b in range(NRING):
            di[b].wait()
            sc.append(pltpu.async_copy(
                ones_v, acc.at[didx.at[b]], sem_s.at[b], add=True))
        for b in range(NRING):
            sc[b].wait()
        return carry

    lax.fori_loop(0, OUTER, round_, 0)

    @pl.when(wid < TAIL)
    def _():
        base = (NWORK * CPW + wid) * C
        pltpu.sync_copy(dst_hbm.at[pl.ds(base, C)], didx.at[0])
        pltpu.sync_copy(ones_v, acc.at[didx.at[0]], add=True)

    plsc.subcore_barrier()
    pltpu.sync_copy(acc.at[pl.ds(row, RPS)], out_hbm.at[c, pl.ds(row, RPS)])


@functools.partial(
    pl.kernel,
    mesh=_MESH,
    out_type=jax.ShapeDtypeStruct((2, NP, NH), jnp.float32),
    scratch_types=[
        pltpu.VMEM((NRING, C), jnp.int32),
        pltpu.VMEM((NRING, C), jnp.int32),
        pltpu.VMEM((NRING, C, NH), jnp.float32),
        pltpu.VMEM_SHARED((NP, NH), jnp.float32),
        pltpu.SemaphoreType.DMA((NRING,)),
        pltpu.SemaphoreType.DMA((NRING,)),
        pltpu.SemaphoreType.DMA((NRING,)),
    ],
    compiler_params=_SC_PARAMS,
)
def _sc_pass(table_hbm, src_hbm, dst_hbm, zeros_hbm, out_hbm,
             sidx, didx, rows, acc, sem_i, sem_g, sem_s):
    c = lax.axis_index("c")
    s = lax.axis_index("s")
    wid = s * 2 + c
    row = s * RPS
    pltpu.sync_copy(zeros_hbm.at[pl.ds(row, RPS)], acc.at[pl.ds(row, RPS)])
    plsc.subcore_barrier()

    def round_(j, carry):
        si = []
        di = []
        for b in range(NRING):
            base = (wid + NWORK * (j * NRING + b)) * C
            si.append(pltpu.async_copy(
                src_hbm.at[pl.ds(base, C)], sidx.at[b], sem_i.at[b]))
            di.append(pltpu.async_copy(
                dst_hbm.at[pl.ds(base, C)], didx.at[b], sem_i.at[b]))
        ga = []
        for b in range(NRING):
            si[b].wait()
            di[b].wait()
            ga.append(pltpu.async_copy(
                table_hbm.at[sidx.at[b]], rows.at[b], sem_g.at[b]))
        sc = []
        for b in range(NRING):
            ga[b].wait()
            sc.append(pltpu.async_copy(
                rows.at[b], acc.at[didx.at[b]], sem_s.at[b], add=True))
        for b in range(NRING):
            sc[b].wait()
        return carry

    lax.fori_loop(0, OUTER, round_, 0)

    @pl.when(wid < TAIL)
    def _():
        base = (NWORK * CPW + wid) * C
        pltpu.sync_copy(src_hbm.at[pl.ds(base, C)], sidx.at[0])
        pltpu.sync_copy(dst_hbm.at[pl.ds(base, C)], didx.at[0])
        pltpu.sync_copy(table_hbm.at[sidx.at[0]], rows.at[0])
        pltpu.sync_copy(rows.at[0], acc.at[didx.at[0]], add=True)

    plsc.subcore_barrier()
    pltpu.sync_copy(acc.at[pl.ds(row, RPS)], out_hbm.at[c, pl.ds(row, RPS)])


_B = 1000


def _dinv_from(dcnt):
    deg = dcnt[0, :, 0] + dcnt[1, :, 0] + 1.0
    return lax.rsqrt(deg)[:, None]


def _xs_body(x_ref, w1_ref, dcnt_ref, xs_ref):
    dinv = _dinv_from(dcnt_ref[...])
    xs_ref[...] = dinv * jnp.dot(x_ref[...], w1_ref[...],
                                 preferred_element_type=jnp.float32)


def _tc_xs(x, w1, dcnt):
    return pl.pallas_call(
        _xs_body,
        grid=(N // _B,),
        in_specs=[
            pl.BlockSpec((_B, NF), lambda i: (i, 0)),
            pl.BlockSpec((NF, NH), lambda i: (0, 0)),
            pl.BlockSpec((2, _B, DEGW), lambda i: (0, i, 0)),
        ],
        out_specs=pl.BlockSpec((_B, NH), lambda i: (i, 0)),
        out_shape=jax.ShapeDtypeStruct((N, NH), jnp.float32),
    )(x, w1, dcnt)


def _hs_body(p_ref, xs_ref, dcnt_ref, b1_ref, wc_ref, hs_ref):
    dinv = _dinv_from(dcnt_ref[...])
    h = dinv * (p_ref[0] + p_ref[1] + xs_ref[...]) + b1_ref[...][None, :]
    hs_ref[...] = dinv * jnp.dot(h, wc_ref[...],
                                 preferred_element_type=jnp.float32)


def _tc_hs(p1, xs, dcnt, b1, wc):
    return pl.pallas_call(
        _hs_body,
        grid=(N // _B,),
        in_specs=[
            pl.BlockSpec((2, _B, NH), lambda i: (0, i, 0)),
            pl.BlockSpec((_B, NH), lambda i: (i, 0)),
            pl.BlockSpec((2, _B, DEGW), lambda i: (0, i, 0)),
            pl.BlockSpec((NH,), lambda i: (0,)),
            pl.BlockSpec((NH, NH), lambda i: (0, 0)),
        ],
        out_specs=pl.BlockSpec((_B, NH), lambda i: (i, 0)),
        out_shape=jax.ShapeDtypeStruct((N, NH), jnp.float32),
    )(p1, xs, dcnt, b1, wc)


def _z_body(p_ref, hs_ref, dcnt_ref, bmu_ref, blv_ref, eps_ref,
            z_ref, mu_ref, lv_ref):
    dinv = _dinv_from(dcnt_ref[...])
    g2 = dinv * (p_ref[0] + p_ref[1] + hs_ref[...])
    mu = g2[:, :NL] + bmu_ref[...][None, :]
    lv = g2[:, NL:] + blv_ref[...][None, :]
    mu_ref[...] = mu
    lv_ref[...] = lv
    z_ref[...] = mu + eps_ref[...] * jnp.exp(0.5 * lv)


def _tc_z(p2, hs, dcnt, bmu, blv, eps):
    return pl.pallas_call(
        _z_body,
        grid=(N // _B,),
        in_specs=[
            pl.BlockSpec((2, _B, NH), lambda i: (0, i, 0)),
            pl.BlockSpec((_B, NH), lambda i: (i, 0)),
            pl.BlockSpec((2, _B, DEGW), lambda i: (0, i, 0)),
            pl.BlockSpec((NL,), lambda i: (0,)),
            pl.BlockSpec((NL,), lambda i: (0,)),
            pl.BlockSpec((_B, NL), lambda i: (i, 0)),
        ],
        out_specs=[
            pl.BlockSpec((_B, NL), lambda i: (i, 0)),
            pl.BlockSpec((_B, NL), lambda i: (i, 0)),
            pl.BlockSpec((_B, NL), lambda i: (i, 0)),
        ],
        out_shape=[
            jax.ShapeDtypeStruct((N, NL), jnp.float32),
            jax.ShapeDtypeStruct((N, NL), jnp.float32),
            jax.ShapeDtypeStruct((N, NL), jnp.float32),
        ],
    )(p2, hs, dcnt, bmu, blv, eps)


def _zm_body(z_ref, zm_ref):
    z = z_ref[...]
    acc = z[0 * ADJ:1 * ADJ] + z[1 * ADJ:2 * ADJ] + z[2 * ADJ:3 * ADJ]
    acc = acc + z[3 * ADJ:4 * ADJ] + z[4 * ADJ:5 * ADJ]
    zm_ref[...] = acc * (1.0 / (N // ADJ))


def _tc_zm(z):
    return pl.pallas_call(
        _zm_body,
        out_shape=jax.ShapeDtypeStruct((ADJ, NL), jnp.float32),
    )(z)


_BA = 1000


def _adj_body(zi_ref, zj_ref, adj_ref):
    prod = lax.dot_general(zi_ref[...], zj_ref[...],
                           (((1,), (1,)), ((), ())),
                           preferred_element_type=jnp.float32)
    adj_ref[...] = jax.nn.sigmoid(prod)


def _tc_adj(zm):
    return pl.pallas_call(
        _adj_body,
        grid=(ADJ // _BA,),
        in_specs=[
            pl.BlockSpec((_BA, NL), lambda i: (i, 0)),
            pl.BlockSpec((ADJ, NL), lambda i: (0, 0)),
        ],
        out_specs=pl.BlockSpec((_BA, ADJ), lambda i: (i, 0)),
        out_shape=jax.ShapeDtypeStruct((ADJ, ADJ), jnp.float32),
    )(zm, zm)


def kernel(x, edge_index, W1, b1, Wmu, bmu, Wlv, blv, eps):
    src = edge_index[0]
    dst = edge_index[1]
    zeros_deg = jnp.zeros((NP, DEGW), jnp.float32)
    ones_c = jnp.ones((C, DEGW), jnp.float32)
    zeros_h = jnp.zeros((NP, NH), jnp.float32)
    wc = jnp.concatenate([Wmu, Wlv], axis=1)

    dcnt = _sc_degree(dst, ones_c, zeros_deg)
    xs = _tc_xs(x, W1, dcnt)
    p1 = _sc_pass(xs, src, dst, zeros_h)
    hs = _tc_hs(p1, xs, dcnt, b1, wc)
    p2 = _sc_pass(hs, src, dst, zeros_h)
    z, mu, lv = _tc_z(p2, hs, dcnt, bmu, blv, eps)
    zm = _tc_zm(z)
    adj = _tc_adj(zm)
    return (z, mu, lv, adj)


# speedup vs baseline: 40.3779x; 1.0749x over previous
import functools

import jax
import jax.numpy as jnp
from jax import lax
from jax.experimental import pallas as pl
from jax.experimental.pallas import tpu as pltpu
from jax.experimental.pallas import tpu_sc as plsc

N = 10000
E = 320000
NF = 128
NH = 64
NL = 32
ADJ = 2000

C = 128
NCHUNK = E // C
NWORK = 32
NRING = 5
CPW = NCHUNK // NWORK
PAIRS = 7
EPILOG = CPW - 2 * PAIRS * NRING - NRING
DRING = 6
DOUTER = CPW // DRING
TAIL = NCHUNK - NWORK * CPW
NSUB = 16
NP = 10240
RPS = NP // NSUB
DEGW = 8

_MESH = plsc.VectorSubcoreMesh(core_axis_name="c", subcore_axis_name="s")
_SC_PARAMS = pltpu.CompilerParams(use_tc_tiling_on_sc=False)


@functools.partial(
    pl.kernel,
    mesh=_MESH,
    out_type=jax.ShapeDtypeStruct((2, NP, DEGW), jnp.float32),
    scratch_types=[
        pltpu.VMEM((DRING, C), jnp.int32),
        pltpu.VMEM((C, DEGW), jnp.float32),
        pltpu.VMEM_SHARED((NP, DEGW), jnp.float32),
        pltpu.SemaphoreType.DMA((DRING,)),
        pltpu.SemaphoreType.DMA((DRING,)),
    ],
    compiler_params=_SC_PARAMS,
)
def _sc_degree(dst_hbm, ones_hbm, zeros_hbm, out_hbm, didx, ones_v, acc,
               sem_i, sem_s):
    c = lax.axis_index("c")
    s = lax.axis_index("s")
    wid = s * 2 + c
    row = s * RPS
    pltpu.sync_copy(zeros_hbm.at[pl.ds(row, RPS)], acc.at[pl.ds(row, RPS)])
    pltpu.sync_copy(ones_hbm, ones_v)
    plsc.subcore_barrier()

    def round_(j, carry):
        di = []
        for b in range(DRING):
            base = (wid + NWORK * (j * DRING + b)) * C
            di.append(pltpu.async_copy(
                dst_hbm.at[pl.ds(base, C)], didx.at[b], sem_i.at[b]))
        sc = []
        for b in range(DRING):
            di[b].wait()
            sc.append(pltpu.async_copy(
                ones_v, acc.at[didx.at[b]], sem_s.at[b], add=True))
        for b in range(DRING):
            sc[b].wait()
        return carry

    lax.fori_loop(0, DOUTER, round_, 0)

    @pl.when(wid < TAIL)
    def _():
        base = (NWORK * CPW + wid) * C
        pltpu.sync_copy(dst_hbm.at[pl.ds(base, C)], didx.at[0])
        pltpu.sync_copy(ones_v, acc.at[didx.at[0]], add=True)

    plsc.subcore_barrier()
    pltpu.sync_copy(acc.at[pl.ds(row, RPS)], out_hbm.at[c, pl.ds(row, RPS)])


@functools.partial(
    pl.kernel,
    mesh=_MESH,
    out_type=jax.ShapeDtypeStruct((2, NP, NH), jnp.float32),
    scratch_types=[
        pltpu.VMEM((2 * NRING, C), jnp.int32),
        pltpu.VMEM((2 * NRING, C), jnp.int32),
        pltpu.VMEM((2 * NRING, C, NH), jnp.float32),
        pltpu.VMEM_SHARED((NP, NH), jnp.float32),
        pltpu.SemaphoreType.DMA((2 * NRING,)),
        pltpu.SemaphoreType.DMA((2 * NRING,)),
        pltpu.SemaphoreType.DMA((2 * NRING,)),
    ],
    compiler_params=_SC_PARAMS,
)
def _sc_pass(table_hbm, src_hbm, dst_hbm, zeros_hbm, out_hbm,
             sidx, didx, rows, acc, sem_i, sem_g, sem_s):
    c = lax.axis_index("c")
    s = lax.axis_index("s")
    wid = s * 2 + c
    row = s * RPS
    pltpu.sync_copy(zeros_hbm.at[pl.ds(row, RPS)], acc.at[pl.ds(row, RPS)])
    plsc.subcore_barrier()

    def two_rounds(jA, jB, countB):
        iA = []
        for q in range(NRING):
            base = (wid + NWORK * (jA * NRING + q)) * C
            iA.append((
                pltpu.async_copy(src_hbm.at[pl.ds(base, C)], sidx.at[q],
                                 sem_i.at[q]),
                pltpu.async_copy(dst_hbm.at[pl.ds(base, C)], didx.at[q],
                                 sem_i.at[q])))
        gA = []
        for q in range(NRING):
            iA[q][0].wait()
            iA[q][1].wait()
            gA.append(pltpu.async_copy(
                table_hbm.at[sidx.at[q]], rows.at[q], sem_g.at[q]))
        iB = []
        for q in range(countB):
            b = NRING + q
            base = (wid + NWORK * (jB * NRING + q)) * C
            iB.append((
                pltpu.async_copy(src_hbm.at[pl.ds(base, C)], sidx.at[b],
                                 sem_i.at[b]),
                pltpu.async_copy(dst_hbm.at[pl.ds(base, C)], didx.at[b],
                                 sem_i.at[b])))
        sA = []
        for q in range(NRING):
            gA[q].wait()
            sA.append(pltpu.async_copy(
                rows.at[q], acc.at[didx.at[q]], sem_s.at[q], add=True))
        gB = []
        for q in range(countB):
            b = NRING + q
            iB[q][0].wait()
            iB[q][1].wait()
            gB.append(pltpu.async_copy(
                table_hbm.at[sidx.at[b]], rows.at[b], sem_g.at[b]))
        for q in range(NRING):
            sA[q].wait()
        sB = []
        for q in range(countB):
            b = NRING + q
            gB[q].wait()
            sB.append(pltpu.async_copy(
                rows.at[b], acc.at[didx.at[b]], sem_s.at[b], add=True))
        for q in range(countB):
            sB[q].wait()

    def pair(jj, carry):
        two_rounds(2 * jj, 2 * jj + 1, NRING)
        return carry

    lax.fori_loop(0, PAIRS, pair, 0)
    two_rounds(2 * PAIRS, 2 * PAIRS + 1, EPILOG)

    @pl.when(wid < TAIL)
    def _():
        base = (NWORK * CPW + wid) * C
        pltpu.sync_copy(dst_hbm.at[pl.ds(base, C)], didx.at[0])
        pltpu.sync_copy(ones_v, acc.at[didx.at[0]], add=True)

    plsc.subcore_barrier()
    pltpu.sync_copy(acc.at[pl.ds(row, RPS)], out_hbm.at[c, pl.ds(row, RPS)])


@functools.partial(
    pl.kernel,
    mesh=_MESH,
    out_type=jax.ShapeDtypeStruct((2, NP, NH), jnp.float32),
    scratch_types=[
        pltpu.VMEM((2 * NRING, C), jnp.int32),
        pltpu.VMEM((2 * NRING, C), jnp.int32),
        pltpu.VMEM((2 * NRING, C, NH), jnp.float32),
        pltpu.VMEM_SHARED((NP, NH), jnp.float32),
        pltpu.SemaphoreType.DMA((2 * NRING,)),
        pltpu.SemaphoreType.DMA((2 * NRING,)),
        pltpu.SemaphoreType.DMA((2 * NRING,)),
    ],
    compiler_params=_SC_PARAMS,
)
def _sc_pass(table_hbm, src_hbm, dst_hbm, zeros_hbm, out_hbm,
             sidx, didx, rows, acc, sem_i, sem_g, sem_s):
    c = lax.axis_index("c")
    s = lax.axis_index("s")
    wid = s * 2 + c
    row = s * RPS
    pltpu.sync_copy(zeros_hbm.at[pl.ds(row, RPS)], acc.at[pl.ds(row, RPS)])
    plsc.subcore_barrier()

    def emit_round(j, ring, drain, count=NRING):
        for q in range(count):
            b = ring * NRING + q
            if drain is True:
                pltpu.make_async_copy(
                    rows.at[b], acc.at[didx.at[b]], sem_s.at[b]).wait()
            elif drain is not False:
                @pl.when(drain)
                def _(b=b):
                    pltpu.make_async_copy(
                        rows.at[b], acc.at[didx.at[b]], sem_s.at[b]).wait()

        ii = []
        for q in range(count):
            b = ring * NRING + q
            base = (wid + NWORK * (j * NRING + q)) * C
            ii.append((
                pltpu.async_copy(src_hbm.at[pl.ds(base, C)], sidx.at[b],
                                 sem_i.at[b]),
                pltpu.async_copy(dst_hbm.at[pl.ds(base, C)], didx.at[b],
                                 sem_i.at[b])))
        ga = []
        for q in range(count):
            b = ring * NRING + q
            ii[q][0].wait()
            ii[q][1].wait()
            ga.append(pltpu.async_copy(
                table_hbm.at[sidx.at[b]], rows.at[b], sem_g.at[b]))
        for q in range(count):
            b = ring * NRING + q
            ga[q].wait()
            pltpu.async_copy(rows.at[b], acc.at[didx.at[b]], sem_s.at[b],
                             add=True)

    def pair(jj, carry):
        emit_round(2 * jj, 0, jj > 0)
        emit_round(2 * jj + 1, 1, jj > 0)
        return carry

    lax.fori_loop(0, PAIRS, pair, 0)
    emit_round(2 * PAIRS, 0, True)
    emit_round(2 * PAIRS + 1, 1, True, count=EPILOG)
    for b in range(2 * NRING):
        pltpu.make_async_copy(
            rows.at[b], acc.at[didx.at[b]], sem_s.at[b]).wait()

    @pl.when(wid < TAIL)
    def _():
        base = (NWORK * CPW + wid) * C
        pltpu.sync_copy(src_hbm.at[pl.ds(base, C)], sidx.at[0])
        pltpu.sync_copy(dst_hbm.at[pl.ds(base, C)], didx.at[0])
        pltpu.sync_copy(table_hbm.at[sidx.at[0]], rows.at[0])
        pltpu.sync_copy(rows.at[0], acc.at[didx.at[0]], add=True)

    plsc.subcore_barrier()
    pltpu.sync_copy(acc.at[pl.ds(row, RPS)], out_hbm.at[c, pl.ds(row, RPS)])


_B = 1000


def _dinv_from(dcnt):
    deg = dcnt[0, :, 0] + dcnt[1, :, 0] + 1.0
    return lax.rsqrt(deg)[:, None]


def _xs_body(x_ref, w1_ref, dcnt_ref, xs_ref):
    dinv = _dinv_from(dcnt_ref[...])
    xs_ref[...] = dinv * jnp.dot(x_ref[...], w1_ref[...],
                                 preferred_element_type=jnp.float32)


def _tc_xs(x, w1, dcnt):
    return pl.pallas_call(
        _xs_body,
        grid=(N // _B,),
        in_specs=[
            pl.BlockSpec((_B, NF), lambda i: (i, 0)),
            pl.BlockSpec((NF, NH), lambda i: (0, 0)),
            pl.BlockSpec((2, _B, DEGW), lambda i: (0, i, 0)),
        ],
        out_specs=pl.BlockSpec((_B, NH), lambda i: (i, 0)),
        out_shape=jax.ShapeDtypeStruct((N, NH), jnp.float32),
    )(x, w1, dcnt)


def _hs_body(p_ref, xs_ref, dcnt_ref, b1_ref, wc_ref, hs_ref):
    dinv = _dinv_from(dcnt_ref[...])
    h = dinv * (p_ref[0] + p_ref[1] + xs_ref[...]) + b1_ref[...][None, :]
    hs_ref[...] = dinv * jnp.dot(h, wc_ref[...],
                                 preferred_element_type=jnp.float32)


def _tc_hs(p1, xs, dcnt, b1, wc):
    return pl.pallas_call(
        _hs_body,
        grid=(N // _B,),
        in_specs=[
            pl.BlockSpec((2, _B, NH), lambda i: (0, i, 0)),
            pl.BlockSpec((_B, NH), lambda i: (i, 0)),
            pl.BlockSpec((2, _B, DEGW), lambda i: (0, i, 0)),
            pl.BlockSpec((NH,), lambda i: (0,)),
            pl.BlockSpec((NH, NH), lambda i: (0, 0)),
        ],
        out_specs=pl.BlockSpec((_B, NH), lambda i: (i, 0)),
        out_shape=jax.ShapeDtypeStruct((N, NH), jnp.float32),
    )(p1, xs, dcnt, b1, wc)


def _z_body(p_ref, hs_ref, dcnt_ref, bmu_ref, blv_ref, eps_ref,
            z_ref, mu_ref, lv_ref):
    dinv = _dinv_from(dcnt_ref[...])
    g2 = dinv * (p_ref[0] + p_ref[1] + hs_ref[...])
    mu = g2[:, :NL] + bmu_ref[...][None, :]
    lv = g2[:, NL:] + blv_ref[...][None, :]
    mu_ref[...] = mu
    lv_ref[...] = lv
    z_ref[...] = mu + eps_ref[...] * jnp.exp(0.5 * lv)


def _tc_z(p2, hs, dcnt, bmu, blv, eps):
    return pl.pallas_call(
        _z_body,
        grid=(N // _B,),
        in_specs=[
            pl.BlockSpec((2, _B, NH), lambda i: (0, i, 0)),
            pl.BlockSpec((_B, NH), lambda i: (i, 0)),
            pl.BlockSpec((2, _B, DEGW), lambda i: (0, i, 0)),
            pl.BlockSpec((NL,), lambda i: (0,)),
            pl.BlockSpec((NL,), lambda i: (0,)),
            pl.BlockSpec((_B, NL), lambda i: (i, 0)),
        ],
        out_specs=[
            pl.BlockSpec((_B, NL), lambda i: (i, 0)),
            pl.BlockSpec((_B, NL), lambda i: (i, 0)),
            pl.BlockSpec((_B, NL), lambda i: (i, 0)),
        ],
        out_shape=[
            jax.ShapeDtypeStruct((N, NL), jnp.float32),
            jax.ShapeDtypeStruct((N, NL), jnp.float32),
            jax.ShapeDtypeStruct((N, NL), jnp.float32),
        ],
    )(p2, hs, dcnt, bmu, blv, eps)


def _zmadj_body(z_ref, adj_ref):
    z = z_ref[...]
    zm = z[0 * ADJ:1 * ADJ] + z[1 * ADJ:2 * ADJ] + z[2 * ADJ:3 * ADJ]
    zm = (zm + z[3 * ADJ:4 * ADJ] + z[4 * ADJ:5 * ADJ]) * (1.0 / (N // ADJ))
    prod = lax.dot_general(zm, zm, (((1,), (1,)), ((), ())),
                           preferred_element_type=jnp.float32)
    adj_ref[...] = jax.nn.sigmoid(prod)


def _tc_zmadj(z):
    return pl.pallas_call(
        _zmadj_body,
        out_shape=jax.ShapeDtypeStruct((ADJ, ADJ), jnp.float32),
    )(z)


def kernel(x, edge_index, W1, b1, Wmu, bmu, Wlv, blv, eps):
    src = edge_index[0]
    dst = edge_index[1]
    zeros_deg = jnp.zeros((NP, DEGW), jnp.float32)
    ones_c = jnp.ones((C, DEGW), jnp.float32)
    zeros_h = jnp.zeros((NP, NH), jnp.float32)
    wc = jnp.concatenate([Wmu, Wlv], axis=1)

    dcnt = _sc_degree(dst, ones_c, zeros_deg)
    xs = _tc_xs(x, W1, dcnt)
    p1 = _sc_pass(xs, src, dst, zeros_h)
    hs = _tc_hs(p1, xs, dcnt, b1, wc)
    p2 = _sc_pass(hs, src, dst, zeros_h)
    z, mu, lv = _tc_z(p2, hs, dcnt, bmu, blv, eps)
    adj = _tc_zmadj(z)
    return (z, mu, lv, adj)


# speedup vs baseline: 43.8723x; 1.0865x over previous
import functools

import jax
import jax.numpy as jnp
from jax import lax
from jax.experimental import pallas as pl
from jax.experimental.pallas import tpu as pltpu
from jax.experimental.pallas import tpu_sc as plsc

N = 10000
E = 320000
NF = 128
NH = 64
NL = 32
ADJ = 2000

C = 128
NCHUNK = E // C
NWORK = 32
NRING = 5
CPW = NCHUNK // NWORK
PAIRS = 7
EPILOG = CPW - 2 * PAIRS * NRING - NRING
DRING = 6
DOUTER = CPW // DRING
TAIL = NCHUNK - NWORK * CPW
NSUB = 16
NP = 10240
RPS = NP // NSUB
DEGW = 8

_MESH = plsc.VectorSubcoreMesh(core_axis_name="c", subcore_axis_name="s")
_SC_PARAMS = pltpu.CompilerParams(use_tc_tiling_on_sc=False)


@functools.partial(
    pl.kernel,
    mesh=_MESH,
    out_type=jax.ShapeDtypeStruct((NP, 128), jnp.float32),
    scratch_types=[
        pltpu.VMEM((DRING, C), jnp.int32),
        pltpu.VMEM((C, DEGW), jnp.float32),
        pltpu.VMEM((C, DEGW), jnp.float32),
        pltpu.VMEM_SHARED((NP, DEGW), jnp.float32),
        pltpu.SemaphoreType.DMA((DRING,)),
        pltpu.SemaphoreType.DMA((DRING,)),
    ],
    compiler_params=_SC_PARAMS,
)
def _sc_degree(dst_hbm, ones_hbm, zeros_hbm, out_hbm, didx, ones_v, zbuf, acc,
               sem_i, sem_s):
    c = lax.axis_index("c")
    s = lax.axis_index("s")
    wid = s * 2 + c
    row = s * RPS
    pltpu.sync_copy(zeros_hbm, zbuf)
    for k in range(RPS // C):
        pltpu.sync_copy(zbuf, acc.at[pl.ds(row + k * C, C)])
    pltpu.sync_copy(ones_hbm, ones_v)
    plsc.subcore_barrier()

    def round_(j, carry):
        di = []
        for b in range(DRING):
            base = (wid + NWORK * (j * DRING + b)) * C
            di.append(pltpu.async_copy(
                dst_hbm.at[pl.ds(base, C)], didx.at[b], sem_i.at[b]))
        sc = []
        for b in range(DRING):
            di[b].wait()
            sc.append(pltpu.async_copy(
                ones_v, acc.at[didx.at[b]], sem_s.at[b], add=True))
        for b in range(DRING):
            sc[b].wait()
        return carry

    lax.fori_loop(0, DOUTER, round_, 0)

    @pl.when(wid < TAIL)
    def _():
        base = (NWORK * CPW + wid) * C
        pltpu.sync_copy(dst_hbm.at[pl.ds(base, C)], didx.at[0])
        pltpu.sync_copy(ones_v, acc.at[didx.at[0]], add=True)

    plsc.subcore_barrier()
    pltpu.sync_copy(acc.at[pl.ds(row, RPS)],
                    out_hbm.at[pl.ds(row, RPS), pl.ds(DEGW * c, DEGW)])


@functools.partial(
    pl.kernel,
    mesh=_MESH,
    out_type=jax.ShapeDtypeStruct((NP, 128), jnp.float32),
    scratch_types=[
        pltpu.VMEM((2 * NRING, C), jnp.int32),
        pltpu.VMEM((2 * NRING, C), jnp.int32),
        pltpu.VMEM((2 * NRING, C, NH), jnp.float32),
        pltpu.VMEM_SHARED((NP, NH), jnp.float32),
        pltpu.SemaphoreType.DMA((2 * NRING,)),
        pltpu.SemaphoreType.DMA((2 * NRING,)),
        pltpu.SemaphoreType.DMA((2 * NRING,)),
    ],
    compiler_params=_SC_PARAMS,
)
def _sc_pass(table_hbm, src_hbm, dst_hbm, zeros_hbm, out_hbm,
             sidx, didx, rows, acc, sem_i, sem_g, sem_s):
    c = lax.axis_index("c")
    s = lax.axis_index("s")
    wid = s * 2 + c
    row = s * RPS
    pltpu.sync_copy(zeros_hbm, rows.at[0])
    for k in range(RPS // C):
        pltpu.sync_copy(rows.at[0], acc.at[pl.ds(row + k * C, C)])
    plsc.subcore_barrier()

    def two_rounds(jA, jB, countB):
        iA = []
        for q in range(NRING):
            base = (wid + NWORK * (jA * NRING + q)) * C
            iA.append((
                pltpu.async_copy(src_hbm.at[pl.ds(base, C)], sidx.at[q],
                                 sem_i.at[q]),
                pltpu.async_copy(dst_hbm.at[pl.ds(base, C)], didx.at[q],
                                 sem_i.at[q])))
        gA = []
        for q in range(NRING):
            iA[q][0].wait()
            iA[q][1].wait()
            gA.append(pltpu.async_copy(
                table_hbm.at[sidx.at[q]], rows.at[q], sem_g.at[q]))
        iB = []
        for q in range(countB):
            b = NRING + q
            base = (wid + NWORK * (jB * NRING + q)) * C
            iB.append((
                pltpu.async_copy(src_hbm.at[pl.ds(base, C)], sidx.at[b],
                                 sem_i.at[b]),
                pltpu.async_copy(dst_hbm.at[pl.ds(base, C)], didx.at[b],
                                 sem_i.at[b])))
        sA = []
        for q in range(NRING):
            gA[q].wait()
            sA.append(pltpu.async_copy(
                rows.at[q], acc.at[didx.at[q]], sem_s.at[q], add=True))
        gB = []
        for q in range(countB):
            b = NRING + q
            iB[q][0].wait()
            iB[q][1].wait()
            gB.append(pltpu.async_copy(
                table_hbm.at[sidx.at[b]], rows.at[b], sem_g.at[b]))
        for q in range(NRING):
            sA[q].wait()
        sB = []
        for q in range(countB):
            b = NRING + q
            gB[q].wait()
            sB.append(pltpu.async_copy(
                rows.at[b], acc.at[didx.at[b]], sem_s.at[b], add=True))
        for q in range(countB):
            sB[q].wait()

    def pair(jj, carry):
        two_rounds(2 * jj, 2 * jj + 1, NRING)
        return carry

    lax.fori_loop(0, PAIRS, pair, 0)
    two_rounds(2 * PAIRS, 2 * PAIRS + 1, EPILOG)

    @pl.when(wid < TAIL)
    def _():
        base = (NWORK * CPW + wid) * C
        pltpu.sync_copy(dst_hbm.at[pl.ds(base, C)], didx.at[0])
        pltpu.sync_copy(ones_v, acc.at[didx.at[0]], add=True)

    plsc.subcore_barrier()
    pltpu.sync_copy(acc.at[pl.ds(row, RPS)],
                    out_hbm.at[pl.ds(row, RPS), pl.ds(DEGW * c, DEGW)])


@functools.partial(
    pl.kernel,
    mesh=_MESH,
    out_type=jax.ShapeDtypeStruct((NP, 128), jnp.float32),
    scratch_types=[
        pltpu.VMEM((2 * NRING, C), jnp.int32),
        pltpu.VMEM((2 * NRING, C), jnp.int32),
        pltpu.VMEM((2 * NRING, C, NH), jnp.float32),
        pltpu.VMEM_SHARED((NP, NH), jnp.float32),
        pltpu.SemaphoreType.DMA((2 * NRING,)),
        pltpu.SemaphoreType.DMA((2 * NRING,)),
        pltpu.SemaphoreType.DMA((2 * NRING,)),
    ],
    compiler_params=_SC_PARAMS,
)
def _sc_pass(table_hbm, src_hbm, dst_hbm, zeros_hbm, out_hbm,
             sidx, didx, rows, acc, sem_i, sem_g, sem_s):
    c = lax.axis_index("c")
    s = lax.axis_index("s")
    wid = s * 2 + c
    row = s * RPS
    pltpu.sync_copy(zeros_hbm, rows.at[0])
    for k in range(RPS // C):
        pltpu.sync_copy(rows.at[0], acc.at[pl.ds(row + k * C, C)])
    plsc.subcore_barrier()

    def emit_round(j, ring, drain, count=NRING):
        for q in range(count):
            b = ring * NRING + q
            if drain is True:
                pltpu.make_async_copy(
                    rows.at[b], acc.at[didx.at[b]], sem_s.at[b]).wait()
            elif drain is not False:
                @pl.when(drain)
                def _(b=b):
                    pltpu.make_async_copy(
                        rows.at[b], acc.at[didx.at[b]], sem_s.at[b]).wait()

        ii = []
        for q in range(count):
            b = ring * NRING + q
            base = (wid + NWORK * (j * NRING + q)) * C
            ii.append((
                pltpu.async_copy(src_hbm.at[pl.ds(base, C)], sidx.at[b],
                                 sem_i.at[b]),
                pltpu.async_copy(dst_hbm.at[pl.ds(base, C)], didx.at[b],
                                 sem_i.at[b])))
        ga = []
        for q in range(count):
            b = ring * NRING + q
            ii[q][0].wait()
            ii[q][1].wait()
            ga.append(pltpu.async_copy(
                table_hbm.at[sidx.at[b]], rows.at[b], sem_g.at[b]))
        for q in range(count):
            b = ring * NRING + q
            ga[q].wait()
            pltpu.async_copy(rows.at[b], acc.at[didx.at[b]], sem_s.at[b],
                             add=True)

    def pair(jj, carry):
        emit_round(2 * jj, 0, jj > 0)
        emit_round(2 * jj + 1, 1, jj > 0)
        return carry

    lax.fori_loop(0, PAIRS, pair, 0)
    emit_round(2 * PAIRS, 0, True)
    emit_round(2 * PAIRS + 1, 1, True, count=EPILOG)
    for b in range(2 * NRING):
        pltpu.make_async_copy(
            rows.at[b], acc.at[didx.at[b]], sem_s.at[b]).wait()

    @pl.when(wid < TAIL)
    def _():
        base = (NWORK * CPW + wid) * C
        pltpu.sync_copy(src_hbm.at[pl.ds(base, C)], sidx.at[0])
        pltpu.sync_copy(dst_hbm.at[pl.ds(base, C)], didx.at[0])
        pltpu.sync_copy(table_hbm.at[sidx.at[0]], rows.at[0])
        pltpu.sync_copy(rows.at[0], acc.at[didx.at[0]], add=True)

    plsc.subcore_barrier()
    pltpu.sync_copy(acc.at[pl.ds(row, RPS)],
                    out_hbm.at[pl.ds(row, RPS), pl.ds(NH * c, NH)])


_B = 1000


def _dinv_from(dcnt):
    deg = dcnt[:, 0] + dcnt[:, DEGW] + 1.0
    return lax.rsqrt(deg)[:, None]


def _xs_body(x_ref, w1_ref, dcnt_ref, xs_ref):
    dinv = _dinv_from(dcnt_ref[...])
    xs_ref[...] = dinv * jnp.dot(x_ref[...], w1_ref[...],
                                 preferred_element_type=jnp.float32)


def _tc_xs(x, w1, dcnt):
    return pl.pallas_call(
        _xs_body,
        grid=(N // _B,),
        in_specs=[
            pl.BlockSpec((_B, NF), lambda i: (i, 0)),
            pl.BlockSpec((NF, NH), lambda i: (0, 0)),
            pl.BlockSpec((_B, 128), lambda i: (i, 0)),
        ],
        out_specs=pl.BlockSpec((_B, NH), lambda i: (i, 0)),
        out_shape=jax.ShapeDtypeStruct((N, NH), jnp.float32),
    )(x, w1, dcnt)


def _hs_body(p_ref, xs_ref, dcnt_ref, b1_ref, wc_ref, hs_ref):
    dinv = _dinv_from(dcnt_ref[...])
    p = p_ref[...]
    h = dinv * (p[:, :NH] + p[:, NH:] + xs_ref[...]) + b1_ref[...][None, :]
    hs_ref[...] = dinv * jnp.dot(h, wc_ref[...],
                                 preferred_element_type=jnp.float32)


def _tc_hs(p1, xs, dcnt, b1, wc):
    return pl.pallas_call(
        _hs_body,
        grid=(N // _B,),
        in_specs=[
            pl.BlockSpec((_B, 128), lambda i: (i, 0)),
            pl.BlockSpec((_B, NH), lambda i: (i, 0)),
            pl.BlockSpec((_B, 128), lambda i: (i, 0)),
            pl.BlockSpec((NH,), lambda i: (0,)),
            pl.BlockSpec((NH, NH), lambda i: (0, 0)),
        ],
        out_specs=pl.BlockSpec((_B, NH), lambda i: (i, 0)),
        out_shape=jax.ShapeDtypeStruct((N, NH), jnp.float32),
    )(p1, xs, dcnt, b1, wc)


def _z_body(p_ref, hs_ref, dcnt_ref, bmu_ref, blv_ref, eps_ref,
            z_ref, mu_ref, lv_ref):
    dinv = _dinv_from(dcnt_ref[...])
    p = p_ref[...]
    g2 = dinv * (p[:, :NH] + p[:, NH:] + hs_ref[...])
    mu = g2[:, :NL] + bmu_ref[...][None, :]
    lv = g2[:, NL:] + blv_ref[...][None, :]
    mu_ref[...] = mu
    lv_ref[...] = lv
    z_ref[...] = mu + eps_ref[...] * jnp.exp(0.5 * lv)


def _tc_z(p2, hs, dcnt, bmu, blv, eps):
    return pl.pallas_call(
        _z_body,
        grid=(N // _B,),
        in_specs=[
            pl.BlockSpec((_B, 128), lambda i: (i, 0)),
            pl.BlockSpec((_B, NH), lambda i: (i, 0)),
            pl.BlockSpec((_B, 128), lambda i: (i, 0)),
            pl.BlockSpec((NL,), lambda i: (0,)),
            pl.BlockSpec((NL,), lambda i: (0,)),
            pl.BlockSpec((_B, NL), lambda i: (i, 0)),
        ],
        out_specs=[
            pl.BlockSpec((_B, NL), lambda i: (i, 0)),
            pl.BlockSpec((_B, NL), lambda i: (i, 0)),
            pl.BlockSpec((_B, NL), lambda i: (i, 0)),
        ],
        out_shape=[
            jax.ShapeDtypeStruct((N, NL), jnp.float32),
            jax.ShapeDtypeStruct((N, NL), jnp.float32),
            jax.ShapeDtypeStruct((N, NL), jnp.float32),
        ],
    )(p2, hs, dcnt, bmu, blv, eps)


def _zmadj_body(z_ref, adj_ref):
    z = z_ref[...]
    zm = z[0 * ADJ:1 * ADJ] + z[1 * ADJ:2 * ADJ] + z[2 * ADJ:3 * ADJ]
    zm = (zm + z[3 * ADJ:4 * ADJ] + z[4 * ADJ:5 * ADJ]) * (1.0 / (N // ADJ))
    prod = lax.dot_general(zm, zm, (((1,), (1,)), ((), ())),
                           preferred_element_type=jnp.float32)
    adj_ref[...] = jax.nn.sigmoid(prod)


def _tc_zmadj(z):
    return pl.pallas_call(
        _zmadj_body,
        out_shape=jax.ShapeDtypeStruct((ADJ, ADJ), jnp.float32),
    )(z)


def kernel(x, edge_index, W1, b1, Wmu, bmu, Wlv, blv, eps):
    src = edge_index[0]
    dst = edge_index[1]
    zeros_deg = jnp.zeros((C, DEGW), jnp.float32)
    ones_c = jnp.ones((C, DEGW), jnp.float32)
    zeros_h = jnp.zeros((C, NH), jnp.float32)
    wc = jnp.concatenate([Wmu, Wlv], axis=1)

    dcnt = _sc_degree(dst, ones_c, zeros_deg)
    xs = _tc_xs(x, W1, dcnt)
    p1 = _sc_pass(xs, src, dst, zeros_h)
    hs = _tc_hs(p1, xs, dcnt, b1, wc)
    p2 = _sc_pass(hs, src, dst, zeros_h)
    z, mu, lv = _tc_z(p2, hs, dcnt, bmu, blv, eps)
    adj = _tc_zmadj(z)
    return (z, mu, lv, adj)


# speedup vs baseline: 46.2127x; 1.0533x over previous
import functools

import jax
import jax.numpy as jnp
from jax import lax
from jax.experimental import pallas as pl
from jax.experimental.pallas import tpu as pltpu
from jax.experimental.pallas import tpu_sc as plsc

N = 10000
E = 320000
NF = 128
NH = 64
NL = 32
ADJ = 2000

C = 128
NCHUNK = E // C
NWORK = 32
NRING = 5
CPW = NCHUNK // NWORK
PAIRS = 7
EPILOG = CPW - 2 * PAIRS * NRING - NRING
DRING = 6
DOUTER = CPW // DRING
TAIL = NCHUNK - NWORK * CPW
NSUB = 16
NP = 10240
RPS = NP // NSUB
DEGW = 8

_MESH = plsc.VectorSubcoreMesh(core_axis_name="c", subcore_axis_name="s")
_SC_PARAMS = pltpu.CompilerParams(use_tc_tiling_on_sc=False)


@functools.partial(
    pl.kernel,
    mesh=_MESH,
    out_type=jax.ShapeDtypeStruct((NP, 128), jnp.float32),
    scratch_types=[
        pltpu.VMEM((DRING, C), jnp.int32),
        pltpu.VMEM((C, DEGW), jnp.float32),
        pltpu.VMEM((C, DEGW), jnp.float32),
        pltpu.VMEM_SHARED((NP, DEGW), jnp.float32),
        pltpu.SemaphoreType.DMA((DRING,)),
        pltpu.SemaphoreType.DMA((DRING,)),
    ],
    compiler_params=_SC_PARAMS,
)
def _sc_degree(ei_hbm, ones_hbm, zeros_hbm, out_hbm, didx, ones_v, zbuf, acc,
               sem_i, sem_s):
    c = lax.axis_index("c")
    s = lax.axis_index("s")
    wid = s * 2 + c
    row = s * RPS
    pltpu.sync_copy(zeros_hbm, zbuf)
    for k in range(RPS // C):
        pltpu.sync_copy(zbuf, acc.at[pl.ds(row + k * C, C)])
    pltpu.sync_copy(ones_hbm, ones_v)
    plsc.subcore_barrier()

    def round_(j, carry):
        di = []
        for b in range(DRING):
            base = (wid + NWORK * (j * DRING + b)) * C
            di.append(pltpu.async_copy(
                ei_hbm.at[1, pl.ds(base, C)], didx.at[b], sem_i.at[b]))
        sc = []
        for b in range(DRING):
            di[b].wait()
            sc.append(pltpu.async_copy(
                ones_v, acc.at[didx.at[b]], sem_s.at[b], add=True))
        for b in range(DRING):
            sc[b].wait()
        return carry

    lax.fori_loop(0, DOUTER, round_, 0)

    @pl.when(wid < TAIL)
    def _():
        base = (NWORK * CPW + wid) * C
        pltpu.sync_copy(ei_hbm.at[1, pl.ds(base, C)], didx.at[0])
        pltpu.sync_copy(ones_v, acc.at[didx.at[0]], add=True)

    plsc.subcore_barrier()
    pltpu.sync_copy(acc.at[pl.ds(row, RPS)],
                    out_hbm.at[pl.ds(row, RPS), pl.ds(DEGW * c, DEGW)])


@functools.partial(
    pl.kernel,
    mesh=_MESH,
    out_type=jax.ShapeDtypeStruct((NP, 128), jnp.float32),
    scratch_types=[
        pltpu.VMEM((2 * NRING, C), jnp.int32),
        pltpu.VMEM((2 * NRING, C), jnp.int32),
        pltpu.VMEM((2 * NRING, C, NH), jnp.float32),
        pltpu.VMEM_SHARED((NP, NH), jnp.float32),
        pltpu.SemaphoreType.DMA((2 * NRING,)),
        pltpu.SemaphoreType.DMA((2 * NRING,)),
        pltpu.SemaphoreType.DMA((2 * NRING,)),
    ],
    compiler_params=_SC_PARAMS,
)
def _sc_pass(table_hbm, ei_hbm, zeros_hbm, out_hbm,
             sidx, didx, rows, acc, sem_i, sem_g, sem_s):
    c = lax.axis_index("c")
    s = lax.axis_index("s")
    wid = s * 2 + c
    row = s * RPS
    pltpu.sync_copy(zeros_hbm, rows.at[0])
    for k in range(RPS // C):
        pltpu.sync_copy(rows.at[0], acc.at[pl.ds(row + k * C, C)])
    plsc.subcore_barrier()

    def two_rounds(jA, jB, countB):
        iA = []
        for q in range(NRING):
            base = (wid + NWORK * (jA * NRING + q)) * C
            iA.append((
                pltpu.async_copy(ei_hbm.at[0, pl.ds(base, C)], sidx.at[q],
                                 sem_i.at[q]),
                pltpu.async_copy(ei_hbm.at[1, pl.ds(base, C)], didx.at[q],
                                 sem_i.at[q])))
        gA = []
        for q in range(NRING):
            iA[q][0].wait()
            iA[q][1].wait()
            gA.append(pltpu.async_copy(
                table_hbm.at[sidx.at[q]], rows.at[q], sem_g.at[q]))
        iB = []
        for q in range(countB):
            b = NRING + q
            base = (wid + NWORK * (jB * NRING + q)) * C
            iB.append((
                pltpu.async_copy(ei_hbm.at[0, pl.ds(base, C)], sidx.at[b],
                                 sem_i.at[b]),
                pltpu.async_copy(ei_hbm.at[1, pl.ds(base, C)], didx.at[b],
                                 sem_i.at[b])))
        sA = []
        for q in range(NRING):
            gA[q].wait()
            sA.append(pltpu.async_copy(
                rows.at[q], acc.at[didx.at[q]], sem_s.at[q], add=True))
        gB = []
        for q in range(countB):
            b = NRING + q
            iB[q][0].wait()
            iB[q][1].wait()
            gB.append(pltpu.async_copy(
                table_hbm.at[sidx.at[b]], rows.at[b], sem_g.at[b]))
        for q in range(NRING):
            sA[q].wait()
        sB = []
        for q in range(countB):
            b = NRING + q
            gB[q].wait()
            sB.append(pltpu.async_copy(
                rows.at[b], acc.at[didx.at[b]], sem_s.at[b], add=True))
        for q in range(countB):
            sB[q].wait()

    def pair(jj, carry):
        two_rounds(2 * jj, 2 * jj + 1, NRING)
        return carry

    lax.fori_loop(0, PAIRS, pair, 0)
    two_rounds(2 * PAIRS, 2 * PAIRS + 1, EPILOG)

    @pl.when(wid < TAIL)
    def _():
        base = (NWORK * CPW + wid) * C
        pltpu.sync_copy(ei_hbm.at[1, pl.ds(base, C)], didx.at[0])
        pltpu.sync_copy(ones_v, acc.at[didx.at[0]], add=True)

    plsc.subcore_barrier()
    pltpu.sync_copy(acc.at[pl.ds(row, RPS)],
                    out_hbm.at[pl.ds(row, RPS), pl.ds(DEGW * c, DEGW)])


@functools.partial(
    pl.kernel,
    mesh=_MESH,
    out_type=jax.ShapeDtypeStruct((NP, 128), jnp.float32),
    scratch_types=[
        pltpu.VMEM((2 * NRING, C), jnp.int32),
        pltpu.VMEM((2 * NRING, C), jnp.int32),
        pltpu.VMEM((2 * NRING, C, NH), jnp.float32),
        pltpu.VMEM_SHARED((NP, NH), jnp.float32),
        pltpu.SemaphoreType.DMA((2 * NRING,)),
        pltpu.SemaphoreType.DMA((2 * NRING,)),
        pltpu.SemaphoreType.DMA((2 * NRING,)),
    ],
    compiler_params=_SC_PARAMS,
)
def _sc_pass(table_hbm, ei_hbm, zeros_hbm, out_hbm,
             sidx, didx, rows, acc, sem_i, sem_g, sem_s):
    c = lax.axis_index("c")
    s = lax.axis_index("s")
    wid = s * 2 + c
    row = s * RPS
    pltpu.sync_copy(zeros_hbm, rows.at[0])
    for k in range(RPS // C):
        pltpu.sync_copy(rows.at[0], acc.at[pl.ds(row + k * C, C)])
    plsc.subcore_barrier()

    def emit_round(j, ring, drain, count=NRING):
        for q in range(count):
            b = ring * NRING + q
            if drain is True:
                pltpu.make_async_copy(
                    rows.at[b], acc.at[didx.at[b]], sem_s.at[b]).wait()
            elif drain is not False:
                @pl.when(drain)
                def _(b=b):
                    pltpu.make_async_copy(
                        rows.at[b], acc.at[didx.at[b]], sem_s.at[b]).wait()

        ii = []
        for q in range(count):
            b = ring * NRING + q
            base = (wid + NWORK * (j * NRING + q)) * C
            ii.append((
                pltpu.async_copy(ei_hbm.at[0, pl.ds(base, C)], sidx.at[b],
                                 sem_i.at[b]),
                pltpu.async_copy(ei_hbm.at[1, pl.ds(base, C)], didx.at[b],
                                 sem_i.at[b])))
        ga = []
        for q in range(count):
            b = ring * NRING + q
            ii[q][0].wait()
            ii[q][1].wait()
            ga.append(pltpu.async_copy(
                table_hbm.at[sidx.at[b]], rows.at[b], sem_g.at[b]))
        for q in range(count):
            b = ring * NRING + q
            ga[q].wait()
            pltpu.async_copy(rows.at[b], acc.at[didx.at[b]], sem_s.at[b],
                             add=True)

    def pair(jj, carry):
        emit_round(2 * jj, 0, jj > 0)
        emit_round(2 * jj + 1, 1, jj > 0)
        return carry

    lax.fori_loop(0, PAIRS, pair, 0)
    emit_round(2 * PAIRS, 0, True)
    emit_round(2 * PAIRS + 1, 1, True, count=EPILOG)
    for b in range(2 * NRING):
        pltpu.make_async_copy(
            rows.at[b], acc.at[didx.at[b]], sem_s.at[b]).wait()

    @pl.when(wid < TAIL)
    def _():
        base = (NWORK * CPW + wid) * C
        pltpu.sync_copy(ei_hbm.at[0, pl.ds(base, C)], sidx.at[0])
        pltpu.sync_copy(ei_hbm.at[1, pl.ds(base, C)], didx.at[0])
        pltpu.sync_copy(table_hbm.at[sidx.at[0]], rows.at[0])
        pltpu.sync_copy(rows.at[0], acc.at[didx.at[0]], add=True)

    plsc.subcore_barrier()
    pltpu.sync_copy(acc.at[pl.ds(row, RPS)],
                    out_hbm.at[pl.ds(row, RPS), pl.ds(NH * c, NH)])


_B = 1000


def _dinv_from(dcnt):
    deg = dcnt[:, 0] + dcnt[:, DEGW] + 1.0
    return lax.rsqrt(deg)[:, None]


def _xs_body(x_ref, w1_ref, dcnt_ref, xs_ref):
    dinv = _dinv_from(dcnt_ref[...])
    xs_ref[...] = dinv * jnp.dot(x_ref[...], w1_ref[...],
                                 preferred_element_type=jnp.float32)


def _tc_xs(x, w1, dcnt):
    return pl.pallas_call(
        _xs_body,
        grid=(N // _B,),
        in_specs=[
            pl.BlockSpec((_B, NF), lambda i: (i, 0)),
            pl.BlockSpec((NF, NH), lambda i: (0, 0)),
            pl.BlockSpec((_B, 128), lambda i: (i, 0)),
        ],
        out_specs=pl.BlockSpec((_B, NH), lambda i: (i, 0)),
        out_shape=jax.ShapeDtypeStruct((N, NH), jnp.float32),
    )(x, w1, dcnt)


def _hs_body(p_ref, xs_ref, dcnt_ref, b1_ref, wc_ref, hs_ref):
    dinv = _dinv_from(dcnt_ref[...])
    p = p_ref[...]
    h = dinv * (p[:, :NH] + p[:, NH:] + xs_ref[...]) + b1_ref[...][None, :]
    hs_ref[...] = dinv * jnp.dot(h, wc_ref[...],
                                 preferred_element_type=jnp.float32)


def _tc_hs(p1, xs, dcnt, b1, wc):
    return pl.pallas_call(
        _hs_body,
        grid=(N // _B,),
        in_specs=[
            pl.BlockSpec((_B, 128), lambda i: (i, 0)),
            pl.BlockSpec((_B, NH), lambda i: (i, 0)),
            pl.BlockSpec((_B, 128), lambda i: (i, 0)),
            pl.BlockSpec((NH,), lambda i: (0,)),
            pl.BlockSpec((NH, NH), lambda i: (0, 0)),
        ],
        out_specs=pl.BlockSpec((_B, NH), lambda i: (i, 0)),
        out_shape=jax.ShapeDtypeStruct((N, NH), jnp.float32),
    )(p1, xs, dcnt, b1, wc)


def _z_body(p_ref, hs_ref, dcnt_ref, bmu_ref, blv_ref, eps_ref,
            z_ref, mu_ref, lv_ref):
    dinv = _dinv_from(dcnt_ref[...])
    p = p_ref[...]
    g2 = dinv * (p[:, :NH] + p[:, NH:] + hs_ref[...])
    mu = g2[:, :NL] + bmu_ref[...][None, :]
    lv = g2[:, NL:] + blv_ref[...][None, :]
    mu_ref[...] = mu
    lv_ref[...] = lv
    z_ref[...] = mu + eps_ref[...] * jnp.exp(0.5 * lv)


def _tc_z(p2, hs, dcnt, bmu, blv, eps):
    return pl.pallas_call(
        _z_body,
        grid=(N // _B,),
        in_specs=[
            pl.BlockSpec((_B, 128), lambda i: (i, 0)),
            pl.BlockSpec((_B, NH), lambda i: (i, 0)),
            pl.BlockSpec((_B, 128), lambda i: (i, 0)),
            pl.BlockSpec((NL,), lambda i: (0,)),
            pl.BlockSpec((NL,), lambda i: (0,)),
            pl.BlockSpec((_B, NL), lambda i: (i, 0)),
        ],
        out_specs=[
            pl.BlockSpec((_B, NL), lambda i: (i, 0)),
            pl.BlockSpec((_B, NL), lambda i: (i, 0)),
            pl.BlockSpec((_B, NL), lambda i: (i, 0)),
        ],
        out_shape=[
            jax.ShapeDtypeStruct((N, NL), jnp.float32),
            jax.ShapeDtypeStruct((N, NL), jnp.float32),
            jax.ShapeDtypeStruct((N, NL), jnp.float32),
        ],
    )(p2, hs, dcnt, bmu, blv, eps)


def _zmadj_body(z_ref, adj_ref):
    z = z_ref[...]
    zm = z[0 * ADJ:1 * ADJ] + z[1 * ADJ:2 * ADJ] + z[2 * ADJ:3 * ADJ]
    zm = (zm + z[3 * ADJ:4 * ADJ] + z[4 * ADJ:5 * ADJ]) * (1.0 / (N // ADJ))
    prod = lax.dot_general(zm, zm, (((1,), (1,)), ((), ())),
                           preferred_element_type=jnp.float32)
    adj_ref[...] = jax.nn.sigmoid(prod)


def _tc_zmadj(z):
    return pl.pallas_call(
        _zmadj_body,
        out_shape=jax.ShapeDtypeStruct((ADJ, ADJ), jnp.float32),
    )(z)


def kernel(x, edge_index, W1, b1, Wmu, bmu, Wlv, blv, eps):
    zeros_deg = jnp.zeros((C, DEGW), jnp.float32)
    ones_c = jnp.ones((C, DEGW), jnp.float32)
    zeros_h = jnp.zeros((C, NH), jnp.float32)
    wc = jnp.concatenate([Wmu, Wlv], axis=1)

    dcnt = _sc_degree(edge_index, ones_c, zeros_deg)
    xs = _tc_xs(x, W1, dcnt)
    p1 = _sc_pass(xs, edge_index, zeros_h)
    hs = _tc_hs(p1, xs, dcnt, b1, wc)
    p2 = _sc_pass(hs, edge_index, zeros_h)
    z, mu, lv = _tc_z(p2, hs, dcnt, bmu, blv, eps)
    adj = _tc_zmadj(z)
    return (z, mu, lv, adj)
